# baseline (speedup 1.0000x reference)
import functools

import jax
import jax.numpy as jnp
import numpy as np
from jax.experimental import pallas as pl
from jax.experimental.pallas import tpu as pltpu

_EPSILON = float(np.finfo(np.float32).tiny)
_K = 2
_TAU = 0.1


def _threefry2x32(k0, k1, x0, x1):
    rot = ((13, 15, 26, 6), (17, 29, 16, 24))
    ks = (np.uint32(k0), np.uint32(k1),
          np.uint32(k0) ^ np.uint32(k1) ^ np.uint32(0x1BD11BDA))
    x0 = x0 + ks[0]
    x1 = x1 + ks[1]
    for i in range(5):
        for r in rot[i % 2]:
            x0 = x0 + x1
            x1 = (x1 << np.uint32(r)) | (x1 >> np.uint32(32 - r))
            x1 = x1 ^ x0
        x0 = x0 + ks[(i + 1) % 3]
        x1 = x1 + ks[(i + 2) % 3] + np.uint32(i + 1)
    return x0, x1


@functools.cache
def _gumbel_noise_t(rows: int, ens: int):
    n = rows * ens
    idx = np.arange(n, dtype=np.uint32)
    o0, o1 = _threefry2x32(0, 1, np.zeros_like(idx), idx)
    bits = o0 ^ o1
    tiny = np.float32(np.finfo(np.float32).tiny)
    fl = ((bits >> np.uint32(9)) | np.uint32(0x3F800000)).view(np.float32)
    fl = fl - np.float32(1.0)
    u = np.maximum(tiny, fl * (np.float32(1.0) - tiny) + tiny)
    g = -np.log(-np.log(u))
    return np.ascontiguousarray(g.reshape(rows, ens).T)


def _softmax_t(y):
    m = jnp.max(y, axis=0, keepdims=True)
    e = jnp.exp(y - m)
    return e * (1.0 / jnp.sum(e, axis=0, keepdims=True))


def _body(x_ref, g_ref, mask_ref, khot_ref):
    x = x_ref[0] + g_ref[...]
    inv_tau = 1.0 / _TAU
    y = x * inv_tau
    a1 = _softmax_t(y)
    y = y + jnp.log(jnp.maximum(1.0 - a1, _EPSILON)) * inv_tau
    a2 = _softmax_t(y)
    khot = a1 + a2

    m1 = jnp.max(khot, axis=0, keepdims=True)
    eq1 = khot == m1
    f1 = jnp.where(eq1, 1.0, 0.0)
    c1 = jnp.sum(f1, axis=0, keepdims=True)
    khot2 = jnp.where(eq1, -jnp.inf, khot)
    m2 = jnp.max(khot2, axis=0, keepdims=True)
    f2 = jnp.where(khot2 == m2, 1.0, 0.0)
    hard = jnp.where(c1 >= 2.0, f1, f1 + f2)

    khot_ref[...] = khot
    mask_ref[0] = hard


def kernel(scores):
    bsz, nmax, ens = scores.shape
    rows = bsz * nmax
    scores_t = jnp.swapaxes(scores, 1, 2)
    g = _gumbel_noise_t(rows, ens)

    blk = 4096 if nmax % 4096 == 0 else nmax
    nblk = nmax // blk
    mask_t, khot_t = pl.pallas_call(
        _body,
        grid=(bsz, nblk),
        in_specs=[
            pl.BlockSpec((1, ens, blk), lambda b, i: (b, 0, i)),
            pl.BlockSpec((ens, blk), lambda b, i: (0, b * nblk + i)),
        ],
        out_specs=[
            pl.BlockSpec((1, ens, blk), lambda b, i: (b, 0, i)),
            pl.BlockSpec((ens, blk), lambda b, i: (0, b * nblk + i)),
        ],
        out_shape=[
            jax.ShapeDtypeStruct((bsz, ens, nmax), jnp.float32),
            jax.ShapeDtypeStruct((ens, rows), jnp.float32),
        ],
        compiler_params=pltpu.CompilerParams(
            dimension_semantics=("parallel", "parallel"),
        ),
    )(scores_t, g)
    return jnp.swapaxes(mask_t, 1, 2), khot_t.T


# speedup vs baseline: 1.2476x; 1.2476x over previous
import functools

import jax
import jax.numpy as jnp
import numpy as np
from jax.experimental import pallas as pl
from jax.experimental.pallas import tpu as pltpu

_EPSILON = float(np.finfo(np.float32).tiny)
_K = 2
_TAU = 0.1


def _threefry2x32(k0, k1, x0, x1):
    rot = ((13, 15, 26, 6), (17, 29, 16, 24))
    ks = (np.uint32(k0), np.uint32(k1),
          np.uint32(k0) ^ np.uint32(k1) ^ np.uint32(0x1BD11BDA))
    x0 = x0 + ks[0]
    x1 = x1 + ks[1]
    for i in range(5):
        for r in rot[i % 2]:
            x0 = x0 + x1
            x1 = (x1 << np.uint32(r)) | (x1 >> np.uint32(32 - r))
            x1 = x1 ^ x0
        x0 = x0 + ks[(i + 1) % 3]
        x1 = x1 + ks[(i + 2) % 3] + np.uint32(i + 1)
    return x0, x1


@functools.cache
def _gumbel_noise_t(rows: int, ens: int):
    n = rows * ens
    idx = np.arange(n, dtype=np.uint32)
    o0, o1 = _threefry2x32(0, 1, np.zeros_like(idx), idx)
    bits = o0 ^ o1
    tiny = np.float32(np.finfo(np.float32).tiny)
    fl = ((bits >> np.uint32(9)) | np.uint32(0x3F800000)).view(np.float32)
    fl = fl - np.float32(1.0)
    u = np.maximum(tiny, fl * (np.float32(1.0) - tiny) + tiny)
    g = -np.log(-np.log(u))
    return np.ascontiguousarray(g.reshape(rows, ens).T)


def _softmax_t(y):
    m = jnp.max(y, axis=0, keepdims=True)
    e = jnp.exp(y - m)
    return e * (1.0 / jnp.sum(e, axis=0, keepdims=True))


def _body(x_ref, g_ref, mask_ref, khot_ref):
    bblk = x_ref.shape[0]
    nmax = x_ref.shape[2]
    inv_tau = 1.0 / _TAU
    for j in range(bblk):
        x = x_ref[j] + g_ref[:, j * nmax:(j + 1) * nmax]
        y = x * inv_tau
        a1 = _softmax_t(y)
        y = y + jnp.log(jnp.maximum(1.0 - a1, _EPSILON)) * inv_tau
        a2 = _softmax_t(y)
        khot = a1 + a2

        m1 = jnp.max(khot, axis=0, keepdims=True)
        eq1 = khot == m1
        f1 = jnp.where(eq1, 1.0, 0.0)
        c1 = jnp.sum(f1, axis=0, keepdims=True)
        khot2 = jnp.where(eq1, -jnp.inf, khot)
        m2 = jnp.max(khot2, axis=0, keepdims=True)
        f2 = jnp.where(khot2 == m2, 1.0, 0.0)
        hard = jnp.where(c1 >= 2.0, f1, f1 + f2)

        khot_ref[:, j * nmax:(j + 1) * nmax] = khot
        mask_ref[j] = hard


def kernel(scores):
    bsz, nmax, ens = scores.shape
    rows = bsz * nmax
    scores_t = jnp.swapaxes(scores, 1, 2)
    g = _gumbel_noise_t(rows, ens)

    bblk = 2 if bsz % 2 == 0 else 1
    mask_t, khot_t = pl.pallas_call(
        _body,
        grid=(bsz // bblk,),
        in_specs=[
            pl.BlockSpec((bblk, ens, nmax), lambda b: (b, 0, 0)),
            pl.BlockSpec((ens, bblk * nmax), lambda b: (0, b)),
        ],
        out_specs=[
            pl.BlockSpec((bblk, ens, nmax), lambda b: (b, 0, 0)),
            pl.BlockSpec((ens, bblk * nmax), lambda b: (0, b)),
        ],
        out_shape=[
            jax.ShapeDtypeStruct((bsz, ens, nmax), jnp.float32),
            jax.ShapeDtypeStruct((ens, rows), jnp.float32),
        ],
        compiler_params=pltpu.CompilerParams(
            dimension_semantics=("parallel",),
        ),
    )(scores_t, g)
    return jnp.swapaxes(mask_t, 1, 2), khot_t.T
